# baseline (speedup 1.0000x reference)
import functools

import numpy as np

import jax
import jax.numpy as jnp
from jax import lax
from jax.experimental import pallas as pl
from jax.experimental.pallas import tpu as pltpu
from jax.experimental.pallas import tpu_sc as plsc

N = 10000
E = 320000
NC = 2
NS = 16
NW = NC * NS
L = 16
K = 128
NCH = 80
HNCH = NCH // 2
EPAD = NW * NCH * K
ACC_ROWS = 10240


def _make_agg(D, depth, nblk):
    mesh = plsc.VectorSubcoreMesh(core_axis_name="c", subcore_axis_name="s")
    BNCH = NCH // nblk

    @functools.partial(
        pl.kernel,
        mesh=mesh,
        compiler_params=pltpu.CompilerParams(use_tc_tiling_on_sc=False),
        out_type=jax.ShapeDtypeStruct((NC, N, D), jnp.float32),
        scratch_types=[
            pltpu.VMEM_SHARED((ACC_ROWS, D), jnp.float32),
            pltpu.VMEM((BNCH, K), jnp.int32),
            pltpu.VMEM((2 * BNCH, K // 2), jnp.int32),
        ] + [pltpu.VMEM((K, D), jnp.float32) for _ in range(depth)]
          + [pltpu.SemaphoreType.DMA for _ in range(depth)],
    )
    def agg(h_hbm, srcp_hbm, dstp_hbm, out_hbm, acc, sidx, didx, *bufs_sems):
        bufs = bufs_sems[:depth]
        sems = bufs_sems[depth:]
        c = lax.axis_index("c")
        s = lax.axis_index("s")
        wid = c * NS + s

        @pl.loop(0, K)
        def _(r):
            @pl.loop(0, D, step=L)
            def _(col):
                bufs[0][r, pl.ds(col, L)] = jnp.zeros((L,), jnp.float32)

        rows_per_tile = ACC_ROWS // NS
        nz = rows_per_tile // K
        for i in range(nz):
            pltpu.async_copy(bufs[0], acc.at[pl.ds(s * rows_per_tile + i * K, K)],
                             sems[i % depth])

        def _drain(q):
            pltpu.make_async_copy(h_hbm.at[pl.ds(0, K)], bufs[q], sems[q]).wait()

        pltpu.sync_copy(srcp_hbm.at[wid, pl.ds(0, BNCH)], sidx)
        pltpu.sync_copy(dstp_hbm.at[wid, pl.ds(0, 2 * BNCH)], didx)
        for i in range(nz):
            _drain(i % depth)
        for q in range(depth):
            pltpu.async_copy(h_hbm.at[sidx.at[q]], bufs[q], sems[q])

        plsc.subcore_barrier()

        for blk in range(nblk):
            if blk > 0:
                pltpu.sync_copy(srcp_hbm.at[wid, pl.ds(blk * BNCH, BNCH)], sidx)
                pltpu.sync_copy(dstp_hbm.at[wid, pl.ds(2 * blk * BNCH, 2 * BNCH)],
                                didx)
                for q in range(depth):
                    pltpu.async_copy(h_hbm.at[sidx.at[q]], bufs[q], sems[q])

            def _scatter(ch, q):
                pltpu.sync_copy(bufs[q].at[pl.ds(0, K // 2)],
                                acc.at[didx.at[2 * ch]], add=True)
                pltpu.sync_copy(bufs[q].at[pl.ds(K // 2, K // 2)],
                                acc.at[didx.at[2 * ch + 1]], add=True)

            @pl.loop(0, BNCH - depth, step=depth)
            def _(j):
                for q in range(depth):
                    _drain(q)
                    _scatter(j + q, q)
                    pltpu.async_copy(h_hbm.at[sidx.at[j + depth + q]],
                                     bufs[q], sems[q])

            for q in range(depth):
                _drain(q)
                _scatter(BNCH - depth + q, q)

        plsc.subcore_barrier()

        out_rows = 624
        pltpu.sync_copy(acc.at[pl.ds(s * out_rows, out_rows)],
                        out_hbm.at[c, pl.ds(s * out_rows, out_rows)])

        @pl.when(s == 0)
        def _():
            pltpu.sync_copy(acc.at[pl.ds(NS * out_rows, N - NS * out_rows)],
                            out_hbm.at[c, pl.ds(NS * out_rows, N - NS * out_rows)])

    return agg


def _make_agg128_k64():
    D = 128
    depth = 4
    nblk = 2
    B2 = (NCH * 2) // nblk
    K2 = 64
    mesh = plsc.VectorSubcoreMesh(core_axis_name="c", subcore_axis_name="s")

    @functools.partial(
        pl.kernel,
        mesh=mesh,
        compiler_params=pltpu.CompilerParams(use_tc_tiling_on_sc=False),
        out_type=jax.ShapeDtypeStruct((NC, N, D), jnp.float32),
        scratch_types=[
            pltpu.VMEM_SHARED((ACC_ROWS, D), jnp.float32),
            pltpu.VMEM((B2 // 2, 2 * K2), jnp.int32),
            pltpu.VMEM((B2, K2), jnp.int32),
        ] + [pltpu.VMEM((K2, D), jnp.float32) for _ in range(depth)]
          + [pltpu.SemaphoreType.DMA for _ in range(depth)],
    )
    def agg(h_hbm, srcp_hbm, dstp_hbm, out_hbm, acc, sidx, didx, *bufs_sems):
        bufs = bufs_sems[:depth]
        sems = bufs_sems[depth:]
        c = lax.axis_index("c")
        s = lax.axis_index("s")
        wid = c * NS + s

        @pl.loop(0, K2)
        def _(r):
            @pl.loop(0, D, step=L)
            def _(col):
                bufs[0][r, pl.ds(col, L)] = jnp.zeros((L,), jnp.float32)

        rows_per_tile = ACC_ROWS // NS
        nz = rows_per_tile // K2
        for i in range(nz):
            pltpu.async_copy(bufs[0], acc.at[pl.ds(s * rows_per_tile + i * K2, K2)],
                             sems[i % depth])

        def _drain(q):
            pltpu.make_async_copy(h_hbm.at[pl.ds(0, K2)], bufs[q], sems[q]).wait()

        def _gather(ch, q):
            idx = sidx.at[ch // 2, pl.ds((q % 2) * K2, K2)]
            pltpu.async_copy(h_hbm.at[idx], bufs[q], sems[q])

        pltpu.sync_copy(srcp_hbm.at[wid, pl.ds(0, B2 // 2)], sidx)
        pltpu.sync_copy(dstp_hbm.at[wid, pl.ds(0, B2)], didx)
        for i in range(nz):
            _drain(i % depth)
        for q in range(depth):
            _gather(q, q)

        plsc.subcore_barrier()

        for blk in range(nblk):
            if blk > 0:
                pltpu.sync_copy(srcp_hbm.at[wid, pl.ds(blk * (B2 // 2), B2 // 2)],
                                sidx)
                pltpu.sync_copy(dstp_hbm.at[wid, pl.ds(blk * B2, B2)], didx)
                for q in range(depth):
                    _gather(q, q)

            @pl.loop(0, B2 - depth, step=depth)
            def _(j):
                for q in range(depth):
                    _drain(q)
                    pltpu.sync_copy(bufs[q], acc.at[didx.at[j + q]], add=True)
                    _gather(j + depth + q, q)

            for q in range(depth):
                _drain(q)
                pltpu.sync_copy(bufs[q], acc.at[didx.at[B2 - depth + q]],
                                add=True)

        plsc.subcore_barrier()

        out_rows = 624
        pltpu.sync_copy(acc.at[pl.ds(s * out_rows, out_rows)],
                        out_hbm.at[c, pl.ds(s * out_rows, out_rows)])

        @pl.when(s == 0)
        def _():
            pltpu.sync_copy(acc.at[pl.ds(NS * out_rows, N - NS * out_rows)],
                            out_hbm.at[c, pl.ds(NS * out_rows, N - NS * out_rows)])

    return agg


_agg128 = _make_agg128_k64()
_agg64 = _make_agg(64, depth=4, nblk=1)

_BM = 2000


def _mm_body(x_ref, w_ref, o_ref):
    o_ref[...] = jnp.dot(x_ref[...], w_ref[...],
                         preferred_element_type=jnp.float32)


def _matmul(x, W):
    M, Kd = x.shape
    D = W.shape[1]
    return pl.pallas_call(
        _mm_body,
        grid=(M // _BM,),
        in_specs=[pl.BlockSpec((_BM, Kd), lambda i: (i, 0)),
                  pl.BlockSpec((Kd, D), lambda i: (0, 0))],
        out_specs=pl.BlockSpec((_BM, D), lambda i: (i, 0)),
        out_shape=jax.ShapeDtypeStruct((M, D), jnp.float32),
    )(x, W)


def _fused_body(p_ref, b_ref, w_ref, o_ref):
    a = jax.nn.relu(p_ref[0] + p_ref[1] + b_ref[...])
    o_ref[...] = jnp.dot(a, w_ref[...],
                         preferred_element_type=jnp.float32)


def _fused_matmul(p, b, W):
    _, M, Kd = p.shape
    D = W.shape[1]
    return pl.pallas_call(
        _fused_body,
        grid=(M // _BM,),
        in_specs=[pl.BlockSpec((2, _BM, Kd), lambda i: (0, i, 0)),
                  pl.BlockSpec((1, Kd), lambda i: (0, 0)),
                  pl.BlockSpec((Kd, D), lambda i: (0, 0))],
        out_specs=pl.BlockSpec((_BM, D), lambda i: (i, 0)),
        out_shape=jax.ShapeDtypeStruct((M, D), jnp.float32),
    )(p, b.reshape(1, Kd), W)


def _combine_body(p_ref, b_ref, o_ref):
    o_ref[...] = p_ref[0] + p_ref[1] + b_ref[...]


def _combine(p, b):
    _, M, D = p.shape
    return pl.pallas_call(
        _combine_body,
        grid=(M // _BM,),
        in_specs=[pl.BlockSpec((2, _BM, D), lambda i: (0, i, 0)),
                  pl.BlockSpec((1, D), lambda i: (0, 0))],
        out_specs=pl.BlockSpec((_BM, D), lambda i: (i, 0)),
        out_shape=jax.ShapeDtypeStruct((M, D), jnp.float32),
    )(p, b.reshape(1, D))


_PAD_SRC = (np.arange(EPAD - E, dtype=np.int32) % N)
_PAD_DST = (N + np.arange(EPAD - E, dtype=np.int32) % (ACC_ROWS - N))


def kernel(x, edge_index, W0, b0, W1, b1, W2, b2):
    ei = edge_index.astype(jnp.int32)
    srcp = jnp.concatenate([ei[0], jnp.asarray(_PAD_SRC)]).reshape(NW, NCH, K)
    dstp = jnp.concatenate([ei[1], jnp.asarray(_PAD_DST)]).reshape(NW, NCH * 2, K // 2)

    h0 = _matmul(x, W0)
    p0 = _agg128(h0, srcp, dstp)
    h1 = _fused_matmul(p0, b0, W1)
    p1 = _agg128(h1, srcp, dstp)
    h2 = _fused_matmul(p1, b1, W2)
    p2 = _agg64(h2, srcp, dstp)
    return _combine(p2, b2)


# speedup vs baseline: 1.0010x; 1.0010x over previous
import functools

import numpy as np

import jax
import jax.numpy as jnp
from jax import lax
from jax.experimental import pallas as pl
from jax.experimental.pallas import tpu as pltpu
from jax.experimental.pallas import tpu_sc as plsc

N = 10000
E = 320000
NC = 2
NS = 16
NW = NC * NS
L = 16
K = 128
NCH = 80
HNCH = NCH // 2
EPAD = NW * NCH * K
ACC_ROWS = 10240


def _make_agg(D, depth, nblk):
    mesh = plsc.VectorSubcoreMesh(core_axis_name="c", subcore_axis_name="s")
    BNCH = NCH // nblk

    @functools.partial(
        pl.kernel,
        mesh=mesh,
        compiler_params=pltpu.CompilerParams(use_tc_tiling_on_sc=False),
        out_type=jax.ShapeDtypeStruct((NC, N, D), jnp.float32),
        scratch_types=[
            pltpu.VMEM_SHARED((ACC_ROWS, D), jnp.float32),
            pltpu.VMEM((BNCH, K), jnp.int32),
            pltpu.VMEM((BNCH, K), jnp.int32),
        ] + [pltpu.VMEM((K, D), jnp.float32) for _ in range(depth)]
          + [pltpu.SemaphoreType.DMA for _ in range(depth)],
    )
    def agg(h_hbm, srcp_hbm, dstp_hbm, out_hbm, acc, sidx, didx, *bufs_sems):
        bufs = bufs_sems[:depth]
        sems = bufs_sems[depth:]
        c = lax.axis_index("c")
        s = lax.axis_index("s")
        wid = c * NS + s

        @pl.loop(0, K)
        def _(r):
            @pl.loop(0, D, step=L)
            def _(col):
                bufs[0][r, pl.ds(col, L)] = jnp.zeros((L,), jnp.float32)

        rows_per_tile = ACC_ROWS // NS
        nz = rows_per_tile // K
        for i in range(nz):
            pltpu.async_copy(bufs[0], acc.at[pl.ds(s * rows_per_tile + i * K, K)],
                             sems[i % depth])

        def _drain(q):
            pltpu.make_async_copy(h_hbm.at[pl.ds(0, K)], bufs[q], sems[q]).wait()

        pltpu.sync_copy(srcp_hbm.at[wid, pl.ds(0, BNCH)], sidx)
        pltpu.sync_copy(dstp_hbm.at[wid, pl.ds(0, BNCH)], didx)
        for i in range(nz):
            _drain(i % depth)
        for q in range(depth):
            pltpu.async_copy(h_hbm.at[sidx.at[q]], bufs[q], sems[q])

        plsc.subcore_barrier()

        for blk in range(nblk):
            if blk > 0:
                pltpu.sync_copy(srcp_hbm.at[wid, pl.ds(blk * BNCH, BNCH)], sidx)
                pltpu.sync_copy(dstp_hbm.at[wid, pl.ds(blk * BNCH, BNCH)], didx)
                for q in range(depth):
                    pltpu.async_copy(h_hbm.at[sidx.at[q]], bufs[q], sems[q])

            @pl.loop(0, BNCH - depth, step=depth)
            def _(j):
                for q in range(depth):
                    _drain(q)
                    pltpu.sync_copy(bufs[q], acc.at[didx.at[j + q]], add=True)
                    pltpu.async_copy(h_hbm.at[sidx.at[j + depth + q]],
                                     bufs[q], sems[q])

            for q in range(depth):
                _drain(q)
                pltpu.sync_copy(bufs[q], acc.at[didx.at[BNCH - depth + q]],
                                add=True)

        plsc.subcore_barrier()

        out_rows = 624
        pltpu.sync_copy(acc.at[pl.ds(s * out_rows, out_rows)],
                        out_hbm.at[c, pl.ds(s * out_rows, out_rows)])

        @pl.when(s == 0)
        def _():
            pltpu.sync_copy(acc.at[pl.ds(NS * out_rows, N - NS * out_rows)],
                            out_hbm.at[c, pl.ds(NS * out_rows, N - NS * out_rows)])

    return agg


def _make_agg128_k64():
    D = 128
    depth = 4
    nblk = 2
    B2 = (NCH * 2) // nblk
    K2 = 64
    mesh = plsc.VectorSubcoreMesh(core_axis_name="c", subcore_axis_name="s")

    @functools.partial(
        pl.kernel,
        mesh=mesh,
        compiler_params=pltpu.CompilerParams(use_tc_tiling_on_sc=False),
        out_type=jax.ShapeDtypeStruct((NC, N, D), jnp.float32),
        scratch_types=[
            pltpu.VMEM_SHARED((ACC_ROWS, D), jnp.float32),
            pltpu.VMEM((B2 // 2, 2 * K2), jnp.int32),
            pltpu.VMEM((B2, K2), jnp.int32),
        ] + [pltpu.VMEM((K2, D), jnp.float32) for _ in range(depth)]
          + [pltpu.SemaphoreType.DMA for _ in range(depth)],
    )
    def agg(h_hbm, srcp_hbm, dstp_hbm, out_hbm, acc, sidx, didx, *bufs_sems):
        bufs = bufs_sems[:depth]
        sems = bufs_sems[depth:]
        c = lax.axis_index("c")
        s = lax.axis_index("s")
        wid = c * NS + s

        @pl.loop(0, K2)
        def _(r):
            @pl.loop(0, D, step=L)
            def _(col):
                bufs[0][r, pl.ds(col, L)] = jnp.zeros((L,), jnp.float32)

        rows_per_tile = ACC_ROWS // NS
        nz = rows_per_tile // K2
        for i in range(nz):
            pltpu.async_copy(bufs[0], acc.at[pl.ds(s * rows_per_tile + i * K2, K2)],
                             sems[i % depth])

        def _drain(q):
            pltpu.make_async_copy(h_hbm.at[pl.ds(0, K2)], bufs[q], sems[q]).wait()

        def _gather(ch, q):
            idx = sidx.at[ch // 2, pl.ds((q % 2) * K2, K2)]
            pltpu.async_copy(h_hbm.at[idx], bufs[q], sems[q])

        pltpu.sync_copy(srcp_hbm.at[wid, pl.ds(0, B2 // 2)], sidx)
        pltpu.sync_copy(dstp_hbm.at[wid, pl.ds(0, B2)], didx)
        for i in range(nz):
            _drain(i % depth)
        for q in range(depth):
            _gather(q, q)

        plsc.subcore_barrier()

        for blk in range(nblk):
            if blk > 0:
                pltpu.sync_copy(srcp_hbm.at[wid, pl.ds(blk * (B2 // 2), B2 // 2)],
                                sidx)
                pltpu.sync_copy(dstp_hbm.at[wid, pl.ds(blk * B2, B2)], didx)
                for q in range(depth):
                    _gather(q, q)

            @pl.loop(0, B2 - depth, step=depth)
            def _(j):
                for q in range(depth):
                    _drain(q)
                    pltpu.sync_copy(bufs[q], acc.at[didx.at[j + q]], add=True)
                    _gather(j + depth + q, q)

            for q in range(depth):
                _drain(q)
                pltpu.sync_copy(bufs[q], acc.at[didx.at[B2 - depth + q]],
                                add=True)

        plsc.subcore_barrier()

        out_rows = 624
        pltpu.sync_copy(acc.at[pl.ds(s * out_rows, out_rows)],
                        out_hbm.at[c, pl.ds(s * out_rows, out_rows)])

        @pl.when(s == 0)
        def _():
            pltpu.sync_copy(acc.at[pl.ds(NS * out_rows, N - NS * out_rows)],
                            out_hbm.at[c, pl.ds(NS * out_rows, N - NS * out_rows)])

    return agg


_agg128 = _make_agg128_k64()
_agg64 = _make_agg(64, depth=4, nblk=1)

_BM = 2000


def _mm_body(x_ref, w_ref, o_ref):
    o_ref[...] = jnp.dot(x_ref[...], w_ref[...],
                         preferred_element_type=jnp.float32)


def _matmul(x, W):
    M, Kd = x.shape
    D = W.shape[1]
    return pl.pallas_call(
        _mm_body,
        grid=(M // _BM,),
        in_specs=[pl.BlockSpec((_BM, Kd), lambda i: (i, 0)),
                  pl.BlockSpec((Kd, D), lambda i: (0, 0))],
        out_specs=pl.BlockSpec((_BM, D), lambda i: (i, 0)),
        out_shape=jax.ShapeDtypeStruct((M, D), jnp.float32),
    )(x, W)


def _fused_body(p_ref, b_ref, w_ref, o_ref):
    a = jax.nn.relu(p_ref[0] + p_ref[1] + b_ref[...])
    o_ref[...] = jnp.dot(a, w_ref[...],
                         preferred_element_type=jnp.float32)


def _fused_matmul(p, b, W):
    _, M, Kd = p.shape
    D = W.shape[1]
    return pl.pallas_call(
        _fused_body,
        grid=(M // _BM,),
        in_specs=[pl.BlockSpec((2, _BM, Kd), lambda i: (0, i, 0)),
                  pl.BlockSpec((1, Kd), lambda i: (0, 0)),
                  pl.BlockSpec((Kd, D), lambda i: (0, 0))],
        out_specs=pl.BlockSpec((_BM, D), lambda i: (i, 0)),
        out_shape=jax.ShapeDtypeStruct((M, D), jnp.float32),
    )(p, b.reshape(1, Kd), W)


def _combine_body(p_ref, b_ref, o_ref):
    o_ref[...] = p_ref[0] + p_ref[1] + b_ref[...]


def _combine(p, b):
    _, M, D = p.shape
    return pl.pallas_call(
        _combine_body,
        grid=(M // _BM,),
        in_specs=[pl.BlockSpec((2, _BM, D), lambda i: (0, i, 0)),
                  pl.BlockSpec((1, D), lambda i: (0, 0))],
        out_specs=pl.BlockSpec((_BM, D), lambda i: (i, 0)),
        out_shape=jax.ShapeDtypeStruct((M, D), jnp.float32),
    )(p, b.reshape(1, D))


_PAD_SRC = (np.arange(EPAD - E, dtype=np.int32) % N)
_PAD_DST = (N + np.arange(EPAD - E, dtype=np.int32) % (ACC_ROWS - N))


def kernel(x, edge_index, W0, b0, W1, b1, W2, b2):
    ei = edge_index.astype(jnp.int32)
    srcp = jnp.concatenate([ei[0], jnp.asarray(_PAD_SRC)]).reshape(NW, NCH, K)
    dstp = jnp.concatenate([ei[1], jnp.asarray(_PAD_DST)]).reshape(NW, NCH * 2, K // 2)

    h0 = _matmul(x, W0)
    p0 = _agg128(h0, srcp, dstp)
    h1 = _fused_matmul(p0, b0, W1)
    p1 = _agg128(h1, srcp, dstp)
    h2 = _fused_matmul(p1, b1, W2)
    p2 = _agg64(h2, srcp, dstp.reshape(NW, NCH, K))
    return _combine(p2, b2)


# speedup vs baseline: 1.0208x; 1.0199x over previous
import functools

import numpy as np

import jax
import jax.numpy as jnp
from jax import lax
from jax.experimental import pallas as pl
from jax.experimental.pallas import tpu as pltpu
from jax.experimental.pallas import tpu_sc as plsc

N = 10000
E = 320000
NC = 2
NS = 16
NW = NC * NS
L = 16
K = 128
NCH = 80
HNCH = NCH // 2
EPAD = NW * NCH * K
ACC_ROWS = 10240


def _make_agg(D, depth, nblk):
    mesh = plsc.VectorSubcoreMesh(core_axis_name="c", subcore_axis_name="s")
    BNCH = NCH // nblk

    @functools.partial(
        pl.kernel,
        mesh=mesh,
        compiler_params=pltpu.CompilerParams(use_tc_tiling_on_sc=False),
        out_type=jax.ShapeDtypeStruct((NC, N, D), jnp.float32),
        scratch_types=[
            pltpu.VMEM_SHARED((ACC_ROWS, D), jnp.float32),
            pltpu.VMEM((BNCH, K), jnp.int32),
            pltpu.VMEM((BNCH, K), jnp.int32),
        ] + [pltpu.VMEM((K, D), jnp.float32) for _ in range(depth)]
          + [pltpu.SemaphoreType.DMA for _ in range(depth)],
    )
    def agg(h_hbm, e3_hbm, out_hbm, acc, sidx, didx, *bufs_sems):
        bufs = bufs_sems[:depth]
        sems = bufs_sems[depth:]
        c = lax.axis_index("c")
        s = lax.axis_index("s")
        wid = c * NS + s

        @pl.loop(0, K)
        def _(r):
            @pl.loop(0, D, step=L)
            def _(col):
                bufs[0][r, pl.ds(col, L)] = jnp.zeros((L,), jnp.float32)

        rows_per_tile = ACC_ROWS // NS
        nz = rows_per_tile // K
        for i in range(nz):
            pltpu.async_copy(bufs[0], acc.at[pl.ds(s * rows_per_tile + i * K, K)],
                             sems[i % depth])

        def _drain(q):
            pltpu.make_async_copy(h_hbm.at[pl.ds(0, K)], bufs[q], sems[q]).wait()

        pltpu.sync_copy(e3_hbm.at[0, wid, pl.ds(0, BNCH)], sidx)
        pltpu.sync_copy(e3_hbm.at[1, wid, pl.ds(0, BNCH)], didx)
        for i in range(nz):
            _drain(i % depth)
        for q in range(depth):
            pltpu.async_copy(h_hbm.at[sidx.at[q]], bufs[q], sems[q])

        plsc.subcore_barrier()

        for blk in range(nblk):
            if blk > 0:
                pltpu.sync_copy(e3_hbm.at[0, wid, pl.ds(blk * BNCH, BNCH)], sidx)
                pltpu.sync_copy(e3_hbm.at[1, wid, pl.ds(blk * BNCH, BNCH)], didx)
                for q in range(depth):
                    pltpu.async_copy(h_hbm.at[sidx.at[q]], bufs[q], sems[q])

            @pl.loop(0, BNCH - depth, step=depth)
            def _(j):
                for q in range(depth):
                    _drain(q)
                    pltpu.sync_copy(bufs[q], acc.at[didx.at[j + q]], add=True)
                    pltpu.async_copy(h_hbm.at[sidx.at[j + depth + q]],
                                     bufs[q], sems[q])

            for q in range(depth):
                _drain(q)
                pltpu.sync_copy(bufs[q], acc.at[didx.at[BNCH - depth + q]],
                                add=True)

        plsc.subcore_barrier()

        out_rows = 624
        pltpu.sync_copy(acc.at[pl.ds(s * out_rows, out_rows)],
                        out_hbm.at[c, pl.ds(s * out_rows, out_rows)])

        @pl.when(s == 0)
        def _():
            pltpu.sync_copy(acc.at[pl.ds(NS * out_rows, N - NS * out_rows)],
                            out_hbm.at[c, pl.ds(NS * out_rows, N - NS * out_rows)])

    return agg


def _make_agg128_k64():
    D = 128
    depth = 4
    nblk = 2
    B2 = (NCH * 2) // nblk
    K2 = 64
    mesh = plsc.VectorSubcoreMesh(core_axis_name="c", subcore_axis_name="s")

    @functools.partial(
        pl.kernel,
        mesh=mesh,
        compiler_params=pltpu.CompilerParams(use_tc_tiling_on_sc=False),
        out_type=jax.ShapeDtypeStruct((NC, N, D), jnp.float32),
        scratch_types=[
            pltpu.VMEM_SHARED((ACC_ROWS, D), jnp.float32),
            pltpu.VMEM((B2 // 2, 2 * K2), jnp.int32),
            pltpu.VMEM((B2, K2), jnp.int32),
        ] + [pltpu.VMEM((K2, D), jnp.float32) for _ in range(depth)]
          + [pltpu.SemaphoreType.DMA for _ in range(depth)],
    )
    def agg(h_hbm, srcp_hbm, dstp_hbm, out_hbm, acc, sidx, didx, *bufs_sems):
        bufs = bufs_sems[:depth]
        sems = bufs_sems[depth:]
        c = lax.axis_index("c")
        s = lax.axis_index("s")
        wid = c * NS + s

        @pl.loop(0, K2)
        def _(r):
            @pl.loop(0, D, step=L)
            def _(col):
                bufs[0][r, pl.ds(col, L)] = jnp.zeros((L,), jnp.float32)

        rows_per_tile = ACC_ROWS // NS
        nz = rows_per_tile // K2
        for i in range(nz):
            pltpu.async_copy(bufs[0], acc.at[pl.ds(s * rows_per_tile + i * K2, K2)],
                             sems[i % depth])

        def _drain(q):
            pltpu.make_async_copy(h_hbm.at[pl.ds(0, K2)], bufs[q], sems[q]).wait()

        def _gather(ch, q):
            idx = sidx.at[ch // 2, pl.ds((q % 2) * K2, K2)]
            pltpu.async_copy(h_hbm.at[idx], bufs[q], sems[q])

        pltpu.sync_copy(srcp_hbm.at[wid, pl.ds(0, B2 // 2)], sidx)
        pltpu.sync_copy(dstp_hbm.at[wid, pl.ds(0, B2)], didx)
        for i in range(nz):
            _drain(i % depth)
        for q in range(depth):
            _gather(q, q)

        plsc.subcore_barrier()

        for blk in range(nblk):
            if blk > 0:
                pltpu.sync_copy(srcp_hbm.at[wid, pl.ds(blk * (B2 // 2), B2 // 2)],
                                sidx)
                pltpu.sync_copy(dstp_hbm.at[wid, pl.ds(blk * B2, B2)], didx)
                for q in range(depth):
                    _gather(q, q)

            @pl.loop(0, B2 - depth, step=depth)
            def _(j):
                for q in range(depth):
                    _drain(q)
                    pltpu.sync_copy(bufs[q], acc.at[didx.at[j + q]], add=True)
                    _gather(j + depth + q, q)

            for q in range(depth):
                _drain(q)
                pltpu.sync_copy(bufs[q], acc.at[didx.at[B2 - depth + q]],
                                add=True)

        plsc.subcore_barrier()

        out_rows = 624
        pltpu.sync_copy(acc.at[pl.ds(s * out_rows, out_rows)],
                        out_hbm.at[c, pl.ds(s * out_rows, out_rows)])

        @pl.when(s == 0)
        def _():
            pltpu.sync_copy(acc.at[pl.ds(NS * out_rows, N - NS * out_rows)],
                            out_hbm.at[c, pl.ds(NS * out_rows, N - NS * out_rows)])

    return agg


_agg128 = _make_agg128_k64()
_agg64 = _make_agg(64, depth=4, nblk=1)

_BM = 2000


def _mm_body(x_ref, w_ref, o_ref):
    o_ref[...] = jnp.dot(x_ref[...], w_ref[...],
                         preferred_element_type=jnp.float32)


def _matmul(x, W):
    M, Kd = x.shape
    D = W.shape[1]
    return pl.pallas_call(
        _mm_body,
        grid=(M // _BM,),
        in_specs=[pl.BlockSpec((_BM, Kd), lambda i: (i, 0)),
                  pl.BlockSpec((Kd, D), lambda i: (0, 0))],
        out_specs=pl.BlockSpec((_BM, D), lambda i: (i, 0)),
        out_shape=jax.ShapeDtypeStruct((M, D), jnp.float32),
    )(x, W)


def _fused_body(p_ref, b_ref, w_ref, o_ref):
    a = jax.nn.relu(p_ref[0] + p_ref[1] + b_ref[...])
    o_ref[...] = jnp.dot(a, w_ref[...],
                         preferred_element_type=jnp.float32)


def _fused_matmul(p, b, W):
    _, M, Kd = p.shape
    D = W.shape[1]
    return pl.pallas_call(
        _fused_body,
        grid=(M // _BM,),
        in_specs=[pl.BlockSpec((2, _BM, Kd), lambda i: (0, i, 0)),
                  pl.BlockSpec((1, Kd), lambda i: (0, 0)),
                  pl.BlockSpec((Kd, D), lambda i: (0, 0))],
        out_specs=pl.BlockSpec((_BM, D), lambda i: (i, 0)),
        out_shape=jax.ShapeDtypeStruct((M, D), jnp.float32),
    )(p, b.reshape(1, Kd), W)


def _combine_body(p_ref, b_ref, o_ref):
    o_ref[...] = p_ref[0] + p_ref[1] + b_ref[...]


def _combine(p, b):
    _, M, D = p.shape
    return pl.pallas_call(
        _combine_body,
        grid=(M // _BM,),
        in_specs=[pl.BlockSpec((2, _BM, D), lambda i: (0, i, 0)),
                  pl.BlockSpec((1, D), lambda i: (0, 0))],
        out_specs=pl.BlockSpec((_BM, D), lambda i: (i, 0)),
        out_shape=jax.ShapeDtypeStruct((M, D), jnp.float32),
    )(p, b.reshape(1, D))


_PAD = np.stack([np.arange(EPAD - E, dtype=np.int32) % N,
                 N + np.arange(EPAD - E, dtype=np.int32) % (ACC_ROWS - N)])


def kernel(x, edge_index, W0, b0, W1, b1, W2, b2):
    e3 = jnp.concatenate([edge_index.astype(jnp.int32), jnp.asarray(_PAD)],
                         axis=1)
    srcp = e3[0].reshape(NW, NCH, K)
    dstp = e3[1].reshape(NW, NCH * 2, K // 2)
    e4 = e3.reshape(2, NW, NCH, K)

    h0 = _matmul(x, W0)
    p0 = _agg128(h0, srcp, dstp)
    h1 = _fused_matmul(p0, b0, W1)
    p1 = _agg128(h1, srcp, dstp)
    h2 = _fused_matmul(p1, b1, W2)
    p2 = _agg64(h2, e4)
    return _combine(p2, b2)


# speedup vs baseline: 1.0214x; 1.0005x over previous
import functools

import numpy as np

import jax
import jax.numpy as jnp
from jax import lax
from jax.experimental import pallas as pl
from jax.experimental.pallas import tpu as pltpu
from jax.experimental.pallas import tpu_sc as plsc

N = 10000
E = 320000
NC = 2
NS = 16
NW = NC * NS
L = 16
K = 128
NCH = 80
EPAD = NW * NCH * K
ACC_ROWS = 10240


def _make_agg(D, depth, nblk):
    mesh = plsc.VectorSubcoreMesh(core_axis_name="c", subcore_axis_name="s")
    BNCH = NCH // nblk

    @functools.partial(
        pl.kernel,
        mesh=mesh,
        compiler_params=pltpu.CompilerParams(use_tc_tiling_on_sc=False),
        out_type=jax.ShapeDtypeStruct((NC, N, D), jnp.float32),
        scratch_types=[
            pltpu.VMEM_SHARED((ACC_ROWS, D), jnp.float32),
            pltpu.VMEM((BNCH, K), jnp.int32),
            pltpu.VMEM((BNCH, K), jnp.int32),
        ] + [pltpu.VMEM((K, D), jnp.float32) for _ in range(depth)]
          + [pltpu.SemaphoreType.DMA for _ in range(depth)],
    )
    def agg(h_hbm, e3_hbm, out_hbm, acc, sidx, didx, *bufs_sems):
        bufs = bufs_sems[:depth]
        sems = bufs_sems[depth:]
        c = lax.axis_index("c")
        s = lax.axis_index("s")
        wid = c * NS + s

        @pl.loop(0, K)
        def _(r):
            @pl.loop(0, D, step=L)
            def _(col):
                bufs[0][r, pl.ds(col, L)] = jnp.zeros((L,), jnp.float32)

        rows_per_tile = ACC_ROWS // NS
        nz = rows_per_tile // K
        for i in range(nz):
            pltpu.async_copy(bufs[0], acc.at[pl.ds(s * rows_per_tile + i * K, K)],
                             sems[i % depth])

        def _drain(q):
            pltpu.make_async_copy(h_hbm.at[pl.ds(0, K)], bufs[q], sems[q]).wait()

        pltpu.sync_copy(e3_hbm.at[0, wid, pl.ds(0, BNCH)], sidx)
        pltpu.sync_copy(e3_hbm.at[1, wid, pl.ds(0, BNCH)], didx)
        for i in range(nz):
            _drain(i % depth)
        for q in range(depth):
            pltpu.async_copy(h_hbm.at[sidx.at[q]], bufs[q], sems[q])

        plsc.subcore_barrier()

        for blk in range(nblk):
            if blk > 0:
                pltpu.sync_copy(e3_hbm.at[0, wid, pl.ds(blk * BNCH, BNCH)], sidx)
                pltpu.sync_copy(e3_hbm.at[1, wid, pl.ds(blk * BNCH, BNCH)], didx)
                for q in range(depth):
                    pltpu.async_copy(h_hbm.at[sidx.at[q]], bufs[q], sems[q])

            @pl.loop(0, BNCH - depth, step=depth)
            def _(j):
                for q in range(depth):
                    _drain(q)
                    pltpu.sync_copy(bufs[q], acc.at[didx.at[j + q]], add=True)
                    pltpu.async_copy(h_hbm.at[sidx.at[j + depth + q]],
                                     bufs[q], sems[q])

            for q in range(depth):
                _drain(q)
                pltpu.sync_copy(bufs[q], acc.at[didx.at[BNCH - depth + q]],
                                add=True)

        plsc.subcore_barrier()

        out_rows = 624
        pltpu.sync_copy(acc.at[pl.ds(s * out_rows, out_rows)],
                        out_hbm.at[c, pl.ds(s * out_rows, out_rows)])

        @pl.when(s == 0)
        def _():
            pltpu.sync_copy(acc.at[pl.ds(NS * out_rows, N - NS * out_rows)],
                            out_hbm.at[c, pl.ds(NS * out_rows, N - NS * out_rows)])

    return agg


def _make_agg128_k64():
    D = 128
    depth = 4
    nblk = 2
    B2 = (NCH * 2) // nblk
    K2 = 64
    mesh = plsc.VectorSubcoreMesh(core_axis_name="c", subcore_axis_name="s")

    @functools.partial(
        pl.kernel,
        mesh=mesh,
        compiler_params=pltpu.CompilerParams(use_tc_tiling_on_sc=False),
        out_type=jax.ShapeDtypeStruct((NC, N, D), jnp.float32),
        scratch_types=[
            pltpu.VMEM_SHARED((ACC_ROWS, D), jnp.float32),
            pltpu.VMEM((B2 // 2, 2 * K2), jnp.int32),
            pltpu.VMEM((B2, K2), jnp.int32),
        ] + [pltpu.VMEM((K2, D), jnp.float32) for _ in range(depth)]
          + [pltpu.SemaphoreType.DMA for _ in range(depth)],
    )
    def agg(h_hbm, srcp_hbm, dstp_hbm, out_hbm, acc, sidx, didx, *bufs_sems):
        bufs = bufs_sems[:depth]
        sems = bufs_sems[depth:]
        c = lax.axis_index("c")
        s = lax.axis_index("s")
        wid = c * NS + s

        @pl.loop(0, K2)
        def _(r):
            @pl.loop(0, D, step=L)
            def _(col):
                bufs[0][r, pl.ds(col, L)] = jnp.zeros((L,), jnp.float32)

        rows_per_tile = ACC_ROWS // NS
        nz = rows_per_tile // K2
        for i in range(nz):
            pltpu.async_copy(bufs[0], acc.at[pl.ds(s * rows_per_tile + i * K2, K2)],
                             sems[i % depth])

        def _drain(q):
            pltpu.make_async_copy(h_hbm.at[pl.ds(0, K2)], bufs[q], sems[q]).wait()

        def _gather(ch, q):
            idx = sidx.at[ch // 2, pl.ds((q % 2) * K2, K2)]
            pltpu.async_copy(h_hbm.at[idx], bufs[q], sems[q])

        pltpu.sync_copy(srcp_hbm.at[wid, pl.ds(0, B2 // 2)], sidx)
        pltpu.sync_copy(dstp_hbm.at[wid, pl.ds(0, B2)], didx)
        for i in range(nz):
            _drain(i % depth)
        for q in range(depth):
            _gather(q, q)

        plsc.subcore_barrier()

        for blk in range(nblk):
            if blk > 0:
                pltpu.sync_copy(srcp_hbm.at[wid, pl.ds(blk * (B2 // 2), B2 // 2)],
                                sidx)
                pltpu.sync_copy(dstp_hbm.at[wid, pl.ds(blk * B2, B2)], didx)
                for q in range(depth):
                    _gather(q, q)

            @pl.loop(0, B2 - depth, step=depth)
            def _(j):
                for q in range(depth):
                    _drain(q)
                    pltpu.sync_copy(bufs[q], acc.at[didx.at[j + q]], add=True)
                    _gather(j + depth + q, q)

            for q in range(depth):
                _drain(q)
                pltpu.sync_copy(bufs[q], acc.at[didx.at[B2 - depth + q]],
                                add=True)

        plsc.subcore_barrier()

        out_rows = 624
        pltpu.sync_copy(acc.at[pl.ds(s * out_rows, out_rows)],
                        out_hbm.at[c, pl.ds(s * out_rows, out_rows)])

        @pl.when(s == 0)
        def _():
            pltpu.sync_copy(acc.at[pl.ds(NS * out_rows, N - NS * out_rows)],
                            out_hbm.at[c, pl.ds(NS * out_rows, N - NS * out_rows)])

    return agg


_agg128 = _make_agg128_k64()
_agg64 = _make_agg(64, depth=4, nblk=1)

_BM = 2000


def _mm_body(x_ref, w_ref, o_ref):
    o_ref[...] = jnp.dot(x_ref[...], w_ref[...],
                         preferred_element_type=jnp.float32)


def _matmul(x, W):
    M, Kd = x.shape
    D = W.shape[1]
    return pl.pallas_call(
        _mm_body,
        grid=(M // _BM,),
        in_specs=[pl.BlockSpec((_BM, Kd), lambda i: (i, 0)),
                  pl.BlockSpec((Kd, D), lambda i: (0, 0))],
        out_specs=pl.BlockSpec((_BM, D), lambda i: (i, 0)),
        out_shape=jax.ShapeDtypeStruct((M, D), jnp.float32),
    )(x, W)


def _fused_body(p_ref, b_ref, w_ref, o_ref):
    a = jax.nn.relu(p_ref[0] + p_ref[1] + b_ref[...])
    o_ref[...] = jnp.dot(a, w_ref[...],
                         preferred_element_type=jnp.float32)


def _fused_matmul(p, b, W):
    _, M, Kd = p.shape
    D = W.shape[1]
    return pl.pallas_call(
        _fused_body,
        grid=(M // _BM,),
        in_specs=[pl.BlockSpec((2, _BM, Kd), lambda i: (0, i, 0)),
                  pl.BlockSpec((1, Kd), lambda i: (0, 0)),
                  pl.BlockSpec((Kd, D), lambda i: (0, 0))],
        out_specs=pl.BlockSpec((_BM, D), lambda i: (i, 0)),
        out_shape=jax.ShapeDtypeStruct((M, D), jnp.float32),
    )(p, b.reshape(1, Kd), W)


def _combine_body(p_ref, b_ref, o_ref):
    o_ref[...] = p_ref[0] + p_ref[1] + b_ref[...]


def _combine(p, b):
    _, M, D = p.shape
    return pl.pallas_call(
        _combine_body,
        grid=(M // _BM,),
        in_specs=[pl.BlockSpec((2, _BM, D), lambda i: (0, i, 0)),
                  pl.BlockSpec((1, D), lambda i: (0, 0))],
        out_specs=pl.BlockSpec((_BM, D), lambda i: (i, 0)),
        out_shape=jax.ShapeDtypeStruct((M, D), jnp.float32),
    )(p, b.reshape(1, D))


_PAD = np.stack([np.arange(EPAD - E, dtype=np.int32) % N,
                 N + np.arange(EPAD - E, dtype=np.int32) % (ACC_ROWS - N)])


def kernel(x, edge_index, W0, b0, W1, b1, W2, b2):
    e3 = jnp.concatenate([edge_index.astype(jnp.int32), jnp.asarray(_PAD)],
                         axis=1)
    srcp = e3[0].reshape(NW, NCH, K)
    dstp = e3[1].reshape(NW, NCH * 2, K // 2)
    e4 = e3.reshape(2, NW, NCH, K)

    h0 = _matmul(x, W0)
    p0 = _agg128(h0, srcp, dstp)
    h1 = _fused_matmul(p0, b0, W1)
    p1 = _agg128(h1, srcp, dstp)
    h2 = _fused_matmul(p1, b1, W2)
    p2 = _agg64(h2, e4)
    return _combine(p2, b2)
